# baseline (speedup 1.0000x reference)
import functools
import math

import jax
import jax.numpy as jnp
from jax.experimental import pallas as pl
from jax.experimental.pallas import tpu as pltpu

T, D, E, K, FF = 2048, 768, 64, 2, 1536
RH = 2 * D
C = int(math.ceil(T * K / E * 1.25))
_HI = jax.lax.Precision.HIGHEST


def _gelu(v):
    return 0.5 * v * (1.0 + jax.lax.erf(v * (1.0 / math.sqrt(2.0))))


def _router_kernel(x_ref, rW1_ref, rb1_ref, rW2_ref, rb2_ref,
                   dtok_ref, dgate_ref, logits_s, rh_s, cnt_s, cols_s):
    CH = 512
    n_ch = T // CH

    def mlp_chunk(i, _):
        sl = pl.ds(i * CH, CH)
        rh_s[...] = jnp.dot(x_ref[sl, :], rW1_ref[...],
                            preferred_element_type=jnp.float32)
        rh = _gelu(rh_s[...] + rb1_ref[...])
        logits_s[sl, :] = jnp.dot(rh, rW2_ref[...],
                                  preferred_element_type=jnp.float32
                                  ) + rb2_ref[...]
        return 0
    jax.lax.fori_loop(0, n_ch, mlp_chunk, 0, unroll=False)

    logits = logits_s[...]
    iota_e = jax.lax.broadcasted_iota(jnp.int32, (T, E), 1)
    m1 = jnp.max(logits, axis=1, keepdims=True)
    idx1 = jnp.min(jnp.where(logits == m1, iota_e, E), axis=1, keepdims=True)
    masked = jnp.where(iota_e == idx1, -jnp.inf, logits)
    m2 = jnp.max(masked, axis=1, keepdims=True)
    idx2 = jnp.min(jnp.where(masked == m2, iota_e, E), axis=1, keepdims=True)
    e2 = jnp.exp(m2 - m1)
    g1 = 1.0 / (1.0 + e2)
    g2 = 1.0 - g1

    oh1 = (iota_e == idx1).astype(jnp.float32)
    oh2 = (iota_e == idx2).astype(jnp.float32)
    cnt_s[...] = oh1 + oh2
    cols_s[:, 0:1] = idx1.astype(jnp.float32)
    cols_s[:, 1:2] = idx2.astype(jnp.float32)
    cols_s[:, 2:3] = g1
    cols_s[:, 3:4] = g2

    BL = 256
    iota_r = jax.lax.broadcasted_iota(jnp.int32, (BL, BL), 0)
    iota_c = jax.lax.broadcasted_iota(jnp.int32, (BL, BL), 1)
    ltri = (iota_r >= iota_c).astype(jnp.float32)

    def cum_chunk(b, carry):
        sl = pl.ds(b * BL, BL)
        blk = cnt_s[sl, :]
        incl = jnp.dot(ltri, blk, preferred_element_type=jnp.float32,
                       precision=_HI)
        cnt_s[sl, :] = incl - blk + carry
        return carry + incl[BL - 1:BL, :]
    jax.lax.fori_loop(0, T // BL, cum_chunk, jnp.zeros((1, E), jnp.float32),
                      unroll=False)

    excl = cnt_s[...]
    pos1 = jnp.sum(excl * oh1, axis=1, keepdims=True)
    pos2 = jnp.sum(excl * oh2, axis=1, keepdims=True) + jnp.sum(
        oh1 * oh2, axis=1, keepdims=True)

    iota_cap = jax.lax.broadcasted_iota(jnp.int32, (T, C), 1).astype(jnp.float32)
    P1 = (pos1 == iota_cap).astype(jnp.float32)
    P2 = (pos2 == iota_cap).astype(jnp.float32)
    tokf = jax.lax.broadcasted_iota(jnp.int32, (T, 1), 0).astype(jnp.float32)
    dn = (((0,), (0,)), ((), ()))
    dtok_f = (jax.lax.dot_general(oh1 * tokf, P1, dn,
                                  preferred_element_type=jnp.float32,
                                  precision=_HI) +
              jax.lax.dot_general(oh2 * tokf, P2, dn,
                                  preferred_element_type=jnp.float32,
                                  precision=_HI))
    dgate_ref[...] = (jax.lax.dot_general(oh1 * g1, P1, dn,
                                          preferred_element_type=jnp.float32,
                                          precision=_HI) +
                      jax.lax.dot_general(oh2 * g2, P2, dn,
                                          preferred_element_type=jnp.float32,
                                          precision=_HI))
    dtok_ref[...] = (dtok_f + 0.5).astype(jnp.int32)


def _expert_kernel(dtok_s, dgate_s, x_ref, eW1_ref, eb1_ref, eW2_ref, eb2_ref,
                   out_ref, xe_s, y_s):
    e = pl.program_id(0)
    j = pl.program_id(1)

    @pl.when(jnp.logical_and(e == 0, j == 0))
    def _init():
        out_ref[...] = jnp.zeros_like(out_ref)

    base = e * C

    @pl.when(j == 0)
    def _gather():
        def gather_body(c, _):
            tok = dtok_s[base + c]
            xe_s[pl.ds(c, 1), :] = x_ref[pl.ds(tok, 1), :]
            return 0
        jax.lax.fori_loop(0, C, gather_body, 0, unroll=False)

    h = jnp.dot(xe_s[...], eW1_ref[0], preferred_element_type=jnp.float32)
    h = _gelu(h + eb1_ref[0])
    yj = jnp.dot(h, eW2_ref[0], preferred_element_type=jnp.float32)

    @pl.when(j == 0)
    def _y0():
        y_s[...] = yj

    @pl.when(j == 1)
    def _y1():
        y_s[...] = y_s[...] + yj + eb2_ref[0]

        def combine_body(c, _):
            tok = dtok_s[base + c]
            g = dgate_s[base + c]
            out_ref[pl.ds(tok, 1), :] = (out_ref[pl.ds(tok, 1), :]
                                         + y_s[pl.ds(c, 1), :] * g)
            return 0
        jax.lax.fori_loop(0, C, combine_body, 0, unroll=False)


@jax.jit
def kernel(x, rW1, rb1, rW2, rb2, eW1, eb1, eW2, eb2):
    dtok, dgate = pl.pallas_call(
        _router_kernel,
        out_shape=[jax.ShapeDtypeStruct((E, C), jnp.int32),
                   jax.ShapeDtypeStruct((E, C), jnp.float32)],
        scratch_shapes=[pltpu.VMEM((T, E), jnp.float32),
                        pltpu.VMEM((512, RH), jnp.float32),
                        pltpu.VMEM((T, E), jnp.float32),
                        pltpu.VMEM((T, 8), jnp.float32)],
    )(x, rW1, rb1.reshape(1, RH), rW2, rb2.reshape(1, E))

    FH = FF // 2
    grid_spec = pltpu.PrefetchScalarGridSpec(
        num_scalar_prefetch=2,
        grid=(E, 2),
        in_specs=[
            pl.BlockSpec((T, D), lambda e, j, s1, s2: (0, 0)),
            pl.BlockSpec((1, D, FH), lambda e, j, s1, s2: (e, 0, j)),
            pl.BlockSpec((1, 1, FH), lambda e, j, s1, s2: (e, 0, j)),
            pl.BlockSpec((1, FH, D), lambda e, j, s1, s2: (e, j, 0)),
            pl.BlockSpec((1, 1, D), lambda e, j, s1, s2: (e, 0, 0)),
        ],
        out_specs=pl.BlockSpec((T, D), lambda e, j, s1, s2: (0, 0)),
        scratch_shapes=[pltpu.VMEM((C, D), jnp.float32),
                        pltpu.VMEM((C, D), jnp.float32)],
    )
    out = pl.pallas_call(
        _expert_kernel,
        grid_spec=grid_spec,
        out_shape=jax.ShapeDtypeStruct((T, D), jnp.float32),
        compiler_params=pltpu.CompilerParams(
            dimension_semantics=("arbitrary", "arbitrary")),
    )(dtok.reshape(-1), dgate.reshape(-1), x,
      eW1, eb1.reshape(E, 1, FF), eW2, eb2.reshape(E, 1, D))
    return out


# speedup vs baseline: 1.1819x; 1.1819x over previous
import functools
import math

import jax
import jax.numpy as jnp
from jax.experimental import pallas as pl
from jax.experimental.pallas import tpu as pltpu

T, D, E, K, FF = 2048, 768, 64, 2, 1536
RH = 2 * D
C = int(math.ceil(T * K / E * 1.25))
_HI = jax.lax.Precision.HIGHEST


def _gelu(v):
    return 0.5 * v * (1.0 + jax.lax.erf(v * (1.0 / math.sqrt(2.0))))


def _router_kernel(x_ref, rW1_ref, rb1_ref, rW2_ref, rb2_ref,
                   dtok_ref, dgate_ref, logits_s, rh_s, cnt_s, cols_s):
    CH = 512
    n_ch = T // CH

    def mlp_chunk(i, _):
        sl = pl.ds(i * CH, CH)
        rh_s[...] = jnp.dot(x_ref[sl, :], rW1_ref[...],
                            preferred_element_type=jnp.float32)
        rh = _gelu(rh_s[...] + rb1_ref[...])
        logits_s[sl, :] = jnp.dot(rh, rW2_ref[...],
                                  preferred_element_type=jnp.float32
                                  ) + rb2_ref[...]
        return 0
    jax.lax.fori_loop(0, n_ch, mlp_chunk, 0, unroll=False)

    logits = logits_s[...]
    iota_e = jax.lax.broadcasted_iota(jnp.int32, (T, E), 1)
    m1 = jnp.max(logits, axis=1, keepdims=True)
    idx1 = jnp.min(jnp.where(logits == m1, iota_e, E), axis=1, keepdims=True)
    masked = jnp.where(iota_e == idx1, -jnp.inf, logits)
    m2 = jnp.max(masked, axis=1, keepdims=True)
    idx2 = jnp.min(jnp.where(masked == m2, iota_e, E), axis=1, keepdims=True)
    e2 = jnp.exp(m2 - m1)
    g1 = 1.0 / (1.0 + e2)
    g2 = 1.0 - g1

    oh1 = (iota_e == idx1).astype(jnp.float32)
    oh2 = (iota_e == idx2).astype(jnp.float32)
    cnt_s[...] = oh1 + oh2
    cols_s[:, 0:1] = idx1.astype(jnp.float32)
    cols_s[:, 1:2] = idx2.astype(jnp.float32)
    cols_s[:, 2:3] = g1
    cols_s[:, 3:4] = g2

    BL = 256
    iota_r = jax.lax.broadcasted_iota(jnp.int32, (BL, BL), 0)
    iota_c = jax.lax.broadcasted_iota(jnp.int32, (BL, BL), 1)
    ltri = (iota_r >= iota_c).astype(jnp.float32)

    def cum_chunk(b, carry):
        sl = pl.ds(b * BL, BL)
        blk = cnt_s[sl, :]
        incl = jnp.dot(ltri, blk, preferred_element_type=jnp.float32,
                       precision=_HI)
        cnt_s[sl, :] = incl - blk + carry
        return carry + incl[BL - 1:BL, :]
    jax.lax.fori_loop(0, T // BL, cum_chunk, jnp.zeros((1, E), jnp.float32),
                      unroll=False)

    excl = cnt_s[...]
    pos1 = jnp.sum(excl * oh1, axis=1, keepdims=True)
    pos2 = jnp.sum(excl * oh2, axis=1, keepdims=True) + jnp.sum(
        oh1 * oh2, axis=1, keepdims=True)

    iota_cap = jax.lax.broadcasted_iota(jnp.int32, (T, C), 1).astype(jnp.float32)
    P1 = (pos1 == iota_cap).astype(jnp.float32)
    P2 = (pos2 == iota_cap).astype(jnp.float32)
    tokf = jax.lax.broadcasted_iota(jnp.int32, (T, 1), 0).astype(jnp.float32)
    dn = (((0,), (0,)), ((), ()))
    dtok_f = (jax.lax.dot_general(oh1 * tokf, P1, dn,
                                  preferred_element_type=jnp.float32,
                                  precision=_HI) +
              jax.lax.dot_general(oh2 * tokf, P2, dn,
                                  preferred_element_type=jnp.float32,
                                  precision=_HI))
    dgate_ref[...] = (jax.lax.dot_general(oh1 * g1, P1, dn,
                                          preferred_element_type=jnp.float32,
                                          precision=_HI) +
                      jax.lax.dot_general(oh2 * g2, P2, dn,
                                          preferred_element_type=jnp.float32,
                                          precision=_HI))
    dtok_ref[...] = (dtok_f + 0.5).astype(jnp.int32)


def _expert_kernel(dtok_s, dgate_s, x_ref, eW1_ref, eb1_ref, eW2_ref, eb2_ref,
                   out_ref, xe_s, y_s):
    e = pl.program_id(0)

    @pl.when(e == 0)
    def _init():
        out_ref[...] = jnp.zeros_like(out_ref)

    base = e * C

    def gather_body(c, _):
        tok = dtok_s[base + c]
        xe_s[pl.ds(c, 1), :] = x_ref[pl.ds(tok, 1), :]
        return 0
    jax.lax.fori_loop(0, C, gather_body, 0, unroll=False)

    h = jnp.dot(xe_s[...], eW1_ref[0], preferred_element_type=jnp.float32)
    h = _gelu(h + eb1_ref[0])
    y_s[...] = jnp.dot(h, eW2_ref[0],
                       preferred_element_type=jnp.float32) + eb2_ref[0]

    def combine_body(c, _):
        tok = dtok_s[base + c]
        g = dgate_s[base + c]
        out_ref[pl.ds(tok, 1), :] = (out_ref[pl.ds(tok, 1), :]
                                     + y_s[pl.ds(c, 1), :] * g)
        return 0
    jax.lax.fori_loop(0, C, combine_body, 0, unroll=False)


@jax.jit
def kernel(x, rW1, rb1, rW2, rb2, eW1, eb1, eW2, eb2):
    dtok, dgate = pl.pallas_call(
        _router_kernel,
        out_shape=[jax.ShapeDtypeStruct((E, C), jnp.int32),
                   jax.ShapeDtypeStruct((E, C), jnp.float32)],
        scratch_shapes=[pltpu.VMEM((T, E), jnp.float32),
                        pltpu.VMEM((512, RH), jnp.float32),
                        pltpu.VMEM((T, E), jnp.float32),
                        pltpu.VMEM((T, 8), jnp.float32)],
    )(x, rW1, rb1.reshape(1, RH), rW2, rb2.reshape(1, E))

    grid_spec = pltpu.PrefetchScalarGridSpec(
        num_scalar_prefetch=2,
        grid=(E,),
        in_specs=[
            pl.BlockSpec((T, D), lambda e, s1, s2: (0, 0)),
            pl.BlockSpec((1, D, FF), lambda e, s1, s2: (e, 0, 0)),
            pl.BlockSpec((1, 1, FF), lambda e, s1, s2: (e, 0, 0)),
            pl.BlockSpec((1, FF, D), lambda e, s1, s2: (e, 0, 0)),
            pl.BlockSpec((1, 1, D), lambda e, s1, s2: (e, 0, 0)),
        ],
        out_specs=pl.BlockSpec((T, D), lambda e, s1, s2: (0, 0)),
        scratch_shapes=[pltpu.VMEM((C, D), jnp.float32),
                        pltpu.VMEM((C, D), jnp.float32)],
    )
    out = pl.pallas_call(
        _expert_kernel,
        grid_spec=grid_spec,
        out_shape=jax.ShapeDtypeStruct((T, D), jnp.float32),
        compiler_params=pltpu.CompilerParams(
            dimension_semantics=("arbitrary",)),
    )(dtok.reshape(-1), dgate.reshape(-1), x,
      eW1, eb1.reshape(E, 1, FF), eW2, eb2.reshape(E, 1, D))
    return out
